# baseline (speedup 1.0000x reference)
import functools

import jax
import jax.numpy as jnp
import numpy as np
from jax import lax
from jax.experimental import pallas as pl
from jax.experimental.pallas import tpu as pltpu
from jax.experimental.pallas import tpu_sc as plsc

_N = 10000
_E = 40000
_P919 = 919
_D = 512
_T = 4
_F = 512
_FO = 128
_EDIM = 50
_NG = 64
_DEG_HIST = np.array([0.0, 500.0, 1000.0, 1500.0, 2000.0, 1800.0, 1200.0, 800.0,
                      500.0, 300.0, 200.0, 100.0, 60.0, 30.0, 10.0])
_AVG_DEG_LOG = float((np.log(np.arange(_DEG_HIST.shape[0]) + 1.0) * _DEG_HIST).sum()
                     / _DEG_HIST.sum())

_PREC = jax.lax.Precision.DEFAULT


def _dot(a, b):
    return jax.lax.dot_general(a, b, (((1,), (0,)), ((), ())), precision=_PREC,
                               preferred_element_type=jnp.float32)


def _kx_body(xe_ref, acts_ref, wact_ref, bact_ref, o_ref):
    o_ref[...] = (xe_ref[...] + _dot(acts_ref[...], wact_ref[...])
                  + bact_ref[...])


def _k_x(xe, acts, W_act, b_act):
    nb = 1000
    return pl.pallas_call(
        _kx_body,
        grid=(_N // nb,),
        in_specs=[
            pl.BlockSpec((nb, _D), lambda i: (i, 0)),
            pl.BlockSpec((nb, 2), lambda i: (i, 0)),
            pl.BlockSpec((2, _D), lambda i: (0, 0)),
            pl.BlockSpec((1, _D), lambda i: (0, 0)),
        ],
        out_specs=pl.BlockSpec((nb, _D), lambda i: (i, 0)),
        out_shape=jax.ShapeDtypeStruct((_N, _D), jnp.float32),
    )(xe, acts, W_act, b_act)


def _kc2_body(ee_ref, we_ref, be_ref, wpe_ref, o_ref):
    e2 = _dot(ee_ref[...], we_ref[...]) + be_ref[...]
    for t in range(_T):
        o_ref[:, t * _F:(t + 1) * _F] = _dot(e2, wpe_ref[t])


def _k_c2(edge_emb, We_l, be_l, Wpre_e_l):
    return pl.pallas_call(
        _kc2_body,
        out_shape=jax.ShapeDtypeStruct((2, _T * _F), jnp.float32),
    )(edge_emb, We_l, be_l.reshape(1, _F), Wpre_e_l)


def _kpre_body(x_ref, wd_ref, ws_ref, bpre_ref, c2_ref, p_ref, t_ref):
    x = x_ref[...]
    p_ref[...] = _dot(x, wd_ref[...]) + bpre_ref[...]
    q = _dot(x, ws_ref[...])
    t_ref[0] = q + c2_ref[0:1, :]
    t_ref[1] = q + c2_ref[1:2, :]


def _k_pre(x, Wd, Ws, bpre_f, C2):
    nb = 400
    return pl.pallas_call(
        _kpre_body,
        grid=(_N // nb,),
        in_specs=[
            pl.BlockSpec((nb, _D), lambda i: (i, 0)),
            pl.BlockSpec((_D, _T * _F), lambda i: (0, 0)),
            pl.BlockSpec((_D, _T * _F), lambda i: (0, 0)),
            pl.BlockSpec((1, _T * _F), lambda i: (0, 0)),
            pl.BlockSpec((2, _T * _F), lambda i: (0, 0)),
        ],
        out_specs=[
            pl.BlockSpec((nb, _T * _F), lambda i: (i, 0)),
            pl.BlockSpec((2, nb, _T * _F), lambda i: (0, i, 0)),
        ],
        out_shape=[
            jax.ShapeDtypeStruct((_N, _T * _F), jnp.float32),
            jax.ShapeDtypeStruct((2, _N, _T * _F), jnp.float32),
        ],
    )(x, Wd, Ws, bpre_f, C2)


def _kpost_body(x_ref, p_ref, s1_ref, s2_ref, mn_ref, mx_ref, deg_ref,
                wpost_ref, bpost_ref, wlin_ref, blin_ref,
                y_ref, bnsum_ref, bnssq_ref):
    i = pl.program_id(0)
    deg = deg_ref[...]
    degc = jnp.maximum(deg, 1.0)
    has = deg > 0.0
    K = p_ref[...]
    m1 = s1_ref[...] / degc
    mean = jnp.where(has, K + m1, 0.0)
    var = jnp.maximum(s2_ref[...] / degc - m1 * m1, 0.0)
    std = jnp.sqrt(var + 1e-5)
    mn = jnp.where(has, K + mn_ref[...], 0.0)
    mx = jnp.where(has, K + mx_ref[...], 0.0)
    ld = jnp.log(degc + 1.0)
    s2c = ld * (1.0 / _AVG_DEG_LOG)
    s3c = _AVG_DEG_LOG / ld
    x = x_ref[...]
    o_parts = []
    for t in range(_T):
        sl = slice(t * _F, (t + 1) * _F)
        agg = jnp.concatenate([mean[:, sl], mn[:, sl], mx[:, sl], std[:, sl]],
                              axis=1)
        post_h = jnp.concatenate([x, agg, agg * s2c, agg * s3c], axis=1)
        o_parts.append(_dot(post_h, wpost_ref[t])
                       + bpost_ref[:, t * _FO:(t + 1) * _FO])
    y = _dot(jnp.concatenate(o_parts, axis=1), wlin_ref[...]) + blin_ref[...]
    y_ref[...] = y

    @pl.when(i == 0)
    def _():
        bnsum_ref[...] = jnp.zeros_like(bnsum_ref)
        bnssq_ref[...] = jnp.zeros_like(bnssq_ref)

    bnsum_ref[...] += jnp.sum(y, axis=0, keepdims=True)
    bnssq_ref[...] += jnp.sum(y * y, axis=0, keepdims=True)


def _k_post(x, P, S1, S2, MN, MX, deg2d, Wpost_l, bpost_f, Wlin_l, blin_f):
    nb = 200
    big = lambda: pl.BlockSpec((nb, _T * _F), lambda i: (i, 0))
    return pl.pallas_call(
        _kpost_body,
        grid=(_N // nb,),
        in_specs=[
            pl.BlockSpec((nb, _D), lambda i: (i, 0)),
            big(), big(), big(), big(), big(),
            pl.BlockSpec((nb, 1), lambda i: (i, 0)),
            pl.BlockSpec((_T, 13 * _F, _FO), lambda i: (0, 0, 0)),
            pl.BlockSpec((1, _T * _FO), lambda i: (0, 0)),
            pl.BlockSpec((_D, _D), lambda i: (0, 0)),
            pl.BlockSpec((1, _D), lambda i: (0, 0)),
        ],
        out_specs=[
            pl.BlockSpec((nb, _D), lambda i: (i, 0)),
            pl.BlockSpec((1, _D), lambda i: (0, 0)),
            pl.BlockSpec((1, _D), lambda i: (0, 0)),
        ],
        out_shape=[
            jax.ShapeDtypeStruct((_N, _D), jnp.float32),
            jax.ShapeDtypeStruct((1, _D), jnp.float32),
            jax.ShapeDtypeStruct((1, _D), jnp.float32),
        ],
    )(x, P, S1, S2, MN, MX, deg2d, Wpost_l, bpost_f, Wlin_l, blin_f)


def _kbn_body(y_ref, sum_ref, ssq_ref, gamma_ref, beta_ref, o_ref):
    m = sum_ref[...] * (1.0 / _N)
    var = ssq_ref[...] * (1.0 / _N) - m * m
    inv = jax.lax.rsqrt(var + 1e-5)
    o_ref[...] = jnp.maximum((y_ref[...] - m) * inv * gamma_ref[...]
                             + beta_ref[...], 0.0)


def _k_bn(y, bnsum, bnssq, gamma_f, beta_f):
    nb = 1000
    return pl.pallas_call(
        _kbn_body,
        grid=(_N // nb,),
        in_specs=[
            pl.BlockSpec((nb, _D), lambda i: (i, 0)),
            pl.BlockSpec((1, _D), lambda i: (0, 0)),
            pl.BlockSpec((1, _D), lambda i: (0, 0)),
            pl.BlockSpec((1, _D), lambda i: (0, 0)),
            pl.BlockSpec((1, _D), lambda i: (0, 0)),
        ],
        out_specs=pl.BlockSpec((nb, _D), lambda i: (i, 0)),
        out_shape=jax.ShapeDtypeStruct((_N, _D), jnp.float32),
    )(y, bnsum, bnssq, gamma_f, beta_f)


def _kpool_body(x_ref, b_ref, o_ref):
    i = pl.program_id(0)

    @pl.when(i == 0)
    def _():
        o_ref[...] = jnp.zeros_like(o_ref)

    b = b_ref[...]
    gid = jax.lax.broadcasted_iota(jnp.int32, (b.shape[0], _NG), 1)
    oh = (b == gid).astype(jnp.float32)
    o_ref[...] += jax.lax.dot_general(oh, x_ref[...], (((0,), (0,)), ((), ())),
                                      precision=_PREC,
                                      preferred_element_type=jnp.float32)


def _k_pool(x, batch2d):
    nb = 1000
    return pl.pallas_call(
        _kpool_body,
        grid=(_N // nb,),
        in_specs=[
            pl.BlockSpec((nb, _D), lambda i: (i, 0)),
            pl.BlockSpec((nb, 1), lambda i: (i, 0)),
        ],
        out_specs=pl.BlockSpec((_NG, _D), lambda i: (0, 0)),
        out_shape=jax.ShapeDtypeStruct((_NG, _D), jnp.float32),
    )(x, batch2d)


def _khead_body(p_ref, wf_ref, bf_ref, wo_ref, bo_ref, a_ref, o_ref):
    h = _dot(p_ref[...], wf_ref[...]) + bf_ref[...]
    a = a_ref[0, 0]
    h = jnp.where(h >= 0.0, h, a * h)
    logits = _dot(h, wo_ref[...]) + bo_ref[...]
    mx = jnp.max(logits, axis=1, keepdims=True)
    lse = jnp.log(jnp.sum(jnp.exp(logits - mx), axis=1, keepdims=True)) + mx
    o_ref[...] = logits - lse


def _k_head(pooled, W_fc1, b_fc1, W_out, b_out, prelu_a):
    return pl.pallas_call(
        _khead_body,
        out_shape=jax.ShapeDtypeStruct((_NG, 2), jnp.float32),
    )(pooled, W_fc1, b_fc1.reshape(1, 2 * _D), W_out, b_out.reshape(1, 2),
      prelu_a.reshape(1, 1).astype(jnp.float32))


_NPS = 320
_NPAD = 10240
_CAP = 2048
_ECH = 4000
_GB = 128
_NB = _CAP // _GB
_NCH = 32
_W = 64


def _sc_mesh():
    return plsc.VectorSubcoreMesh(core_axis_name="c", subcore_axis_name="s")


_SC_CP = pltpu.CompilerParams(use_tc_tiling_on_sc=False,
                              needs_layout_passes=False)


def _embed_gather(node_emb, gidx_pad):
    @functools.partial(
        pl.kernel, mesh=_sc_mesh(), compiler_params=_SC_CP,
        out_type=jax.ShapeDtypeStruct((_NPAD, _D), jnp.float32),
        scratch_types=[
            pltpu.VMEM((80,), jnp.int32),
            pltpu.VMEM((80, _D), jnp.float32),
            pltpu.SemaphoreType.DMA,
        ],
    )
    def k(tab_hbm, idx_hbm, out_hbm, idx_v, rows_v, sem):
        wid = lax.axis_index("s") * 2 + lax.axis_index("c")
        base = wid * _NPS

        @pl.loop(0, 4)
        def _(j):
            b = base + j * 80
            pltpu.sync_copy(idx_hbm.at[pl.ds(b, 80)], idx_v)
            pltpu.async_copy(tab_hbm.at[idx_v], rows_v, sem).wait()
            pltpu.sync_copy(rows_v, out_hbm.at[pl.ds(b, 80)])

    return k(node_emb, gidx_pad)


def _filter_edges(src, dst, sgn):
    f32, i32 = jnp.float32, jnp.int32
    out_type = [
        jax.ShapeDtypeStruct((32, _CAP), i32),
        jax.ShapeDtypeStruct((32, _CAP * 16), i32),
        jax.ShapeDtypeStruct((_NPAD,), f32),
    ]

    @functools.partial(
        pl.kernel, mesh=_sc_mesh(), out_type=out_type,
        compiler_params=_SC_CP,
        scratch_types=[
            pltpu.VMEM((_ECH,), i32),
            pltpu.VMEM((_ECH,), i32),
            pltpu.VMEM((_ECH,), i32),
            pltpu.VMEM((4096,), i32),
            pltpu.VMEM((4096,), i32),
            pltpu.VMEM((_CAP,), i32),
            pltpu.VMEM((_CAP,), i32),
            pltpu.VMEM((_CAP * 16,), i32),
            pltpu.VMEM((_NPS + 16,), f32),
            pltpu.VMEM((16,), i32),
            pltpu.SMEM((1,), i32),
            pltpu.SMEM((1,), i32),
        ],
    )
    def k(src_hbm, dst_hbm, sgn_hbm, lei_hbm, lsp_hbm, deg_hbm,
          esrc, edst, esgn, eflat, eldv, leidx, ldst, ldsplat, dacc, mbuf,
          cnt_s, off_s):
        wid = lax.axis_index("s") * 2 + lax.axis_index("c")
        base = wid * _NPS
        iota = lax.iota(i32, 16)
        zeros16 = jnp.zeros((16,), f32)
        ones16 = jnp.ones((16,), f32)
        zi16 = jnp.zeros((16,), i32)
        lane0 = iota == 0

        @pl.loop(0, _CAP, step=16)
        def _(i):
            leidx[pl.ds(i, 16)] = zi16
            ldst[pl.ds(i, 16)] = jnp.full((16,), _NPS, i32)

        @pl.loop(0, _CAP * 16, step=16)
        def _(i):
            ldsplat[pl.ds(i, 16)] = jnp.full((16,), _NPS, i32)

        @pl.loop(0, _ECH, step=16)
        def _(i):
            eflat[pl.ds(i, 16)] = zi16
            eldv[pl.ds(i, 16)] = zi16

        @pl.loop(0, _NPS + 16, step=16)
        def _(i):
            dacc[pl.ds(i, 16)] = zeros16

        cnt_s[0] = jnp.int32(0)
        mbuf[...] = jnp.full((16,), 1, i32)

        @pl.loop(0, _E // _ECH)
        def _(ck):
            off = ck * _ECH
            pltpu.sync_copy(src_hbm.at[pl.ds(off, _ECH)], esrc)
            pltpu.sync_copy(dst_hbm.at[pl.ds(off, _ECH)], edst)
            pltpu.sync_copy(sgn_hbm.at[pl.ds(off, _ECH)], esgn)

            @pl.loop(0, _ECH, step=16)
            def _(i):
                off_s[0] = i
                ii = off_s[0]
                tmsk = mbuf[...] >= 1
                plsc.store_scatter(
                    eflat, [ii + iota],
                    esrc[pl.ds(i, 16)] + esgn[pl.ds(i, 16)] * _N, mask=tmsk)
                plsc.store_scatter(
                    eldv, [ii + iota],
                    edst[pl.ds(i, 16)] - base, mask=tmsk)

            @pl.loop(0, _ECH)
            def _(i):
                iv = jnp.full((16,), i, i32)
                ev = plsc.load_gather(eflat, [iv])
                ldv = plsc.load_gather(eldv, [iv])
                cnt = cnt_s[0]
                keep = (ldv >= 0) & (ldv < _NPS) & (cnt < _CAP)
                m = lane0 & keep
                posv = jnp.full((16,), cnt, i32)
                plsc.store_scatter(leidx, [posv], ev, mask=m)
                plsc.store_scatter(ldst, [posv], ldv, mask=m)
                plsc.store_scatter(ldsplat, [posv * 16 + iota], ldv,
                                  mask=keep)
                cnt_s[0] = cnt + jnp.sum(m.astype(i32))

        @pl.loop(0, _CAP)
        def _(i):
            iv = jnp.full((16,), i, i32)
            lv = plsc.load_gather(ldst, [iv])
            plsc.addupdate_scatter(dacc, [lv], ones16, mask=lane0)

        pltpu.sync_copy(dacc.at[pl.ds(0, _NPS)], deg_hbm.at[pl.ds(base, _NPS)])
        pltpu.sync_copy(leidx, lei_hbm.at[wid])
        pltpu.sync_copy(ldsplat, lsp_hbm.at[wid])

    return k(src, dst, sgn)


def _seg_accum(tab_flat, lei, lsp):
    f32, i32 = jnp.float32, jnp.int32
    out_type = [jax.ShapeDtypeStruct((_NPAD, _T * _F), f32) for _ in range(4)]

    @functools.partial(
        pl.kernel, mesh=_sc_mesh(), out_type=out_type,
        compiler_params=_SC_CP,
        scratch_types=[
            pltpu.VMEM((_CAP,), i32),
            pltpu.VMEM((_CAP * 16,), i32),
            pltpu.VMEM((_GB,), i32),
            pltpu.VMEM((_NPS + 1, _W), f32),
            pltpu.VMEM((_NPS + 1, _W), f32),
            pltpu.VMEM((_NPS + 1, _W), f32),
            pltpu.VMEM((_NPS + 1, _W), f32),
            pltpu.VMEM((_GB, _W), f32),
            pltpu.SemaphoreType.DMA,
        ],
    )
    def k(tab_hbm, lei_hbm, lsp_hbm,
          s1_hbm, s2_hbm, mn_hbm, mx_hbm,
          leidx, ldsplat, idxbuf, s1a, s2a, mna, mxa, gbuf, sem):
        wid = lax.axis_index("s") * 2 + lax.axis_index("c")
        base = wid * _NPS
        iota = lax.iota(i32, 16)
        zeros16 = jnp.zeros((16,), f32)

        pltpu.sync_copy(lei_hbm.at[wid], leidx)
        pltpu.sync_copy(lsp_hbm.at[wid], ldsplat)

        @pl.loop(0, _NCH)
        def _(c):
            @pl.loop(0, _NPS + 1)
            def _(r):
                for kk in range(_W // 16):
                    sl = pl.ds(kk * 16, 16)
                    s1a[r, sl] = zeros16
                    s2a[r, sl] = zeros16
                    mna[r, sl] = jnp.full((16,), 3.0e38, f32)
                    mxa[r, sl] = jnp.full((16,), -3.0e38, f32)

            @pl.loop(0, _NB)
            def _(b):
                @pl.loop(0, _GB, step=16)
                def _(j):
                    ev = leidx[pl.ds(b * _GB + j, 16)]
                    idxbuf[pl.ds(j, 16)] = ev * _NCH + c

                pltpu.async_copy(tab_hbm.at[idxbuf], gbuf, sem).wait()

                @pl.loop(0, _GB)
                def _(i):
                    ldv = ldsplat[pl.ds((b * _GB + i) * 16, 16)]
                    for kk in range(_W // 16):
                        iok = iota + kk * 16
                        row = gbuf[i, pl.ds(kk * 16, 16)]
                        plsc.addupdate_scatter(s1a, [ldv, iok], row)
                        plsc.addupdate_scatter(s2a, [ldv, iok], row * row)
                        cmn = plsc.load_gather(mna, [ldv, iok])
                        plsc.store_scatter(mna, [ldv, iok],
                                           jnp.minimum(cmn, row))
                        cmx = plsc.load_gather(mxa, [ldv, iok])
                        plsc.store_scatter(mxa, [ldv, iok],
                                           jnp.maximum(cmx, row))

            cs = pl.ds(c * _W, _W)
            rs = pl.ds(0, _NPS)
            pltpu.sync_copy(s1a.at[rs], s1_hbm.at[pl.ds(base, _NPS), cs])
            pltpu.sync_copy(s2a.at[rs], s2_hbm.at[pl.ds(base, _NPS), cs])
            pltpu.sync_copy(mna.at[rs], mn_hbm.at[pl.ds(base, _NPS), cs])
            pltpu.sync_copy(mxa.at[rs], mx_hbm.at[pl.ds(base, _NPS), cs])

    return k(tab_flat, lei, lsp)


def kernel(global_idx, acts, sign, edge_index, batch, node_emb, edge_emb,
           W_act, b_act, We, be, Wpre, bpre, Wpost, bpost, Wlin, blin,
           gamma, beta, W_fc1, b_fc1, W_out, b_out, prelu_a):
    src = edge_index[0].astype(jnp.int32)
    dst = edge_index[1].astype(jnp.int32)
    sgn = sign.astype(jnp.int32)

    gidx_pad = jnp.concatenate([global_idx.astype(jnp.int32),
                                jnp.zeros((_NPAD - _N,), jnp.int32)])
    xe = _embed_gather(node_emb, gidx_pad)
    x = _k_x(xe, acts, W_act, b_act.reshape(1, _D))

    lei, lds, deg = _filter_edges(src, dst, sgn)
    deg2d = deg.reshape(-1, 1)

    for l in range(2):
        Wd = Wpre[l][:, :_F, :].transpose(1, 0, 2).reshape(_F, _T * _F)
        Ws = Wpre[l][:, _F:2 * _F, :].transpose(1, 0, 2).reshape(_F, _T * _F)
        Wpre_e = Wpre[l][:, 2 * _F:, :]
        bpre_f = bpre[l].reshape(1, _T * _F)
        C2 = _k_c2(edge_emb, We[l], be[l], Wpre_e)
        P, Tbl = _k_pre(x, Wd, Ws, bpre_f, C2)
        S1, S2, MN, MX = _seg_accum(
            Tbl.reshape(2 * _N * _NCH, _W), lei, lds)
        y, bnsum, bnssq = _k_post(x, P, S1, S2, MN, MX, deg2d,
                                  Wpost[l], bpost[l].reshape(1, _T * _FO),
                                  Wlin[l], blin[l].reshape(1, _D))
        x = _k_bn(y, bnsum, bnssq, gamma[l].reshape(1, _D),
                  beta[l].reshape(1, _D))

    pooled = _k_pool(x, batch.astype(jnp.int32).reshape(-1, 1))
    return _k_head(pooled, W_fc1, b_fc1, W_out, b_out, prelu_a)


# speedup vs baseline: 1.7835x; 1.7835x over previous
import functools

import jax
import jax.numpy as jnp
import numpy as np
from jax import lax
from jax.experimental import pallas as pl
from jax.experimental.pallas import tpu as pltpu
from jax.experimental.pallas import tpu_sc as plsc

_N = 10000
_E = 40000
_P919 = 919
_D = 512
_T = 4
_F = 512
_FO = 128
_EDIM = 50
_NG = 64
_DEG_HIST = np.array([0.0, 500.0, 1000.0, 1500.0, 2000.0, 1800.0, 1200.0, 800.0,
                      500.0, 300.0, 200.0, 100.0, 60.0, 30.0, 10.0])
_AVG_DEG_LOG = float((np.log(np.arange(_DEG_HIST.shape[0]) + 1.0) * _DEG_HIST).sum()
                     / _DEG_HIST.sum())

_PREC = jax.lax.Precision.DEFAULT


def _dot(a, b):
    return jax.lax.dot_general(a, b, (((1,), (0,)), ((), ())), precision=_PREC,
                               preferred_element_type=jnp.float32)


def _kx_body(xe_ref, acts_ref, wact_ref, bact_ref, o_ref):
    o_ref[...] = (xe_ref[...] + _dot(acts_ref[...], wact_ref[...])
                  + bact_ref[...])


def _k_x(xe, acts, W_act, b_act):
    nb = 1000
    return pl.pallas_call(
        _kx_body,
        grid=(_N // nb,),
        in_specs=[
            pl.BlockSpec((nb, _D), lambda i: (i, 0)),
            pl.BlockSpec((nb, 2), lambda i: (i, 0)),
            pl.BlockSpec((2, _D), lambda i: (0, 0)),
            pl.BlockSpec((1, _D), lambda i: (0, 0)),
        ],
        out_specs=pl.BlockSpec((nb, _D), lambda i: (i, 0)),
        out_shape=jax.ShapeDtypeStruct((_N, _D), jnp.float32),
    )(xe, acts, W_act, b_act)


def _kc2_body(ee_ref, we_ref, be_ref, wpe_ref, o_ref):
    e2 = _dot(ee_ref[...], we_ref[...]) + be_ref[...]
    for t in range(_T):
        o_ref[:, t * _F:(t + 1) * _F] = _dot(e2, wpe_ref[t])


def _k_c2(edge_emb, We_l, be_l, Wpre_e_l):
    return pl.pallas_call(
        _kc2_body,
        out_shape=jax.ShapeDtypeStruct((2, _T * _F), jnp.float32),
    )(edge_emb, We_l, be_l.reshape(1, _F), Wpre_e_l)


def _kpre_body(x_ref, wd_ref, ws_ref, bpre_ref, c2_ref, p_ref, t_ref):
    x = x_ref[...]
    p_ref[...] = _dot(x, wd_ref[...]) + bpre_ref[...]
    q = _dot(x, ws_ref[...])
    t_ref[0] = q + c2_ref[0:1, :]
    t_ref[1] = q + c2_ref[1:2, :]


def _k_pre(x, Wd, Ws, bpre_f, C2):
    nb = 400
    return pl.pallas_call(
        _kpre_body,
        grid=(_N // nb,),
        in_specs=[
            pl.BlockSpec((nb, _D), lambda i: (i, 0)),
            pl.BlockSpec((_D, _T * _F), lambda i: (0, 0)),
            pl.BlockSpec((_D, _T * _F), lambda i: (0, 0)),
            pl.BlockSpec((1, _T * _F), lambda i: (0, 0)),
            pl.BlockSpec((2, _T * _F), lambda i: (0, 0)),
        ],
        out_specs=[
            pl.BlockSpec((nb, _T * _F), lambda i: (i, 0)),
            pl.BlockSpec((2, nb, _T * _F), lambda i: (0, i, 0)),
        ],
        out_shape=[
            jax.ShapeDtypeStruct((_N, _T * _F), jnp.float32),
            jax.ShapeDtypeStruct((2, _N, _T * _F), jnp.float32),
        ],
    )(x, Wd, Ws, bpre_f, C2)


def _kpost_body(x_ref, p_ref, s1_ref, s2_ref, mn_ref, mx_ref, deg_ref,
                wpost_ref, bpost_ref, wlin_ref, blin_ref,
                y_ref, bnsum_ref, bnssq_ref):
    i = pl.program_id(0)
    deg = deg_ref[...]
    degc = jnp.maximum(deg, 1.0)
    has = deg > 0.0
    K = p_ref[...]
    m1 = s1_ref[...] / degc
    mean = jnp.where(has, K + m1, 0.0)
    var = jnp.maximum(s2_ref[...] / degc - m1 * m1, 0.0)
    std = jnp.sqrt(var + 1e-5)
    mn = jnp.where(has, K + mn_ref[...], 0.0)
    mx = jnp.where(has, K + mx_ref[...], 0.0)
    ld = jnp.log(degc + 1.0)
    s2c = ld * (1.0 / _AVG_DEG_LOG)
    s3c = _AVG_DEG_LOG / ld
    x = x_ref[...]
    o_parts = []
    for t in range(_T):
        sl = slice(t * _F, (t + 1) * _F)
        agg = jnp.concatenate([mean[:, sl], mn[:, sl], mx[:, sl], std[:, sl]],
                              axis=1)
        post_h = jnp.concatenate([x, agg, agg * s2c, agg * s3c], axis=1)
        o_parts.append(_dot(post_h, wpost_ref[t])
                       + bpost_ref[:, t * _FO:(t + 1) * _FO])
    y = _dot(jnp.concatenate(o_parts, axis=1), wlin_ref[...]) + blin_ref[...]
    y_ref[...] = y

    @pl.when(i == 0)
    def _():
        bnsum_ref[...] = jnp.zeros_like(bnsum_ref)
        bnssq_ref[...] = jnp.zeros_like(bnssq_ref)

    bnsum_ref[...] += jnp.sum(y, axis=0, keepdims=True)
    bnssq_ref[...] += jnp.sum(y * y, axis=0, keepdims=True)


def _k_post(x, P, S1, S2, MN, MX, deg2d, Wpost_l, bpost_f, Wlin_l, blin_f):
    nb = 200
    big = lambda: pl.BlockSpec((nb, _T * _F), lambda i: (i, 0))
    return pl.pallas_call(
        _kpost_body,
        grid=(_N // nb,),
        in_specs=[
            pl.BlockSpec((nb, _D), lambda i: (i, 0)),
            big(), big(), big(), big(), big(),
            pl.BlockSpec((nb, 1), lambda i: (i, 0)),
            pl.BlockSpec((_T, 13 * _F, _FO), lambda i: (0, 0, 0)),
            pl.BlockSpec((1, _T * _FO), lambda i: (0, 0)),
            pl.BlockSpec((_D, _D), lambda i: (0, 0)),
            pl.BlockSpec((1, _D), lambda i: (0, 0)),
        ],
        out_specs=[
            pl.BlockSpec((nb, _D), lambda i: (i, 0)),
            pl.BlockSpec((1, _D), lambda i: (0, 0)),
            pl.BlockSpec((1, _D), lambda i: (0, 0)),
        ],
        out_shape=[
            jax.ShapeDtypeStruct((_N, _D), jnp.float32),
            jax.ShapeDtypeStruct((1, _D), jnp.float32),
            jax.ShapeDtypeStruct((1, _D), jnp.float32),
        ],
    )(x, P, S1, S2, MN, MX, deg2d, Wpost_l, bpost_f, Wlin_l, blin_f)


def _kbn_body(y_ref, sum_ref, ssq_ref, gamma_ref, beta_ref, o_ref):
    m = sum_ref[...] * (1.0 / _N)
    var = ssq_ref[...] * (1.0 / _N) - m * m
    inv = jax.lax.rsqrt(var + 1e-5)
    o_ref[...] = jnp.maximum((y_ref[...] - m) * inv * gamma_ref[...]
                             + beta_ref[...], 0.0)


def _k_bn(y, bnsum, bnssq, gamma_f, beta_f):
    nb = 1000
    return pl.pallas_call(
        _kbn_body,
        grid=(_N // nb,),
        in_specs=[
            pl.BlockSpec((nb, _D), lambda i: (i, 0)),
            pl.BlockSpec((1, _D), lambda i: (0, 0)),
            pl.BlockSpec((1, _D), lambda i: (0, 0)),
            pl.BlockSpec((1, _D), lambda i: (0, 0)),
            pl.BlockSpec((1, _D), lambda i: (0, 0)),
        ],
        out_specs=pl.BlockSpec((nb, _D), lambda i: (i, 0)),
        out_shape=jax.ShapeDtypeStruct((_N, _D), jnp.float32),
    )(y, bnsum, bnssq, gamma_f, beta_f)


def _kpool_body(x_ref, b_ref, o_ref):
    i = pl.program_id(0)

    @pl.when(i == 0)
    def _():
        o_ref[...] = jnp.zeros_like(o_ref)

    b = b_ref[...]
    gid = jax.lax.broadcasted_iota(jnp.int32, (b.shape[0], _NG), 1)
    oh = (b == gid).astype(jnp.float32)
    o_ref[...] += jax.lax.dot_general(oh, x_ref[...], (((0,), (0,)), ((), ())),
                                      precision=_PREC,
                                      preferred_element_type=jnp.float32)


def _k_pool(x, batch2d):
    nb = 1000
    return pl.pallas_call(
        _kpool_body,
        grid=(_N // nb,),
        in_specs=[
            pl.BlockSpec((nb, _D), lambda i: (i, 0)),
            pl.BlockSpec((nb, 1), lambda i: (i, 0)),
        ],
        out_specs=pl.BlockSpec((_NG, _D), lambda i: (0, 0)),
        out_shape=jax.ShapeDtypeStruct((_NG, _D), jnp.float32),
    )(x, batch2d)


def _khead_body(p_ref, wf_ref, bf_ref, wo_ref, bo_ref, a_ref, o_ref):
    h = _dot(p_ref[...], wf_ref[...]) + bf_ref[...]
    a = a_ref[0, 0]
    h = jnp.where(h >= 0.0, h, a * h)
    logits = _dot(h, wo_ref[...]) + bo_ref[...]
    mx = jnp.max(logits, axis=1, keepdims=True)
    lse = jnp.log(jnp.sum(jnp.exp(logits - mx), axis=1, keepdims=True)) + mx
    o_ref[...] = logits - lse


def _k_head(pooled, W_fc1, b_fc1, W_out, b_out, prelu_a):
    return pl.pallas_call(
        _khead_body,
        out_shape=jax.ShapeDtypeStruct((_NG, 2), jnp.float32),
    )(pooled, W_fc1, b_fc1.reshape(1, 2 * _D), W_out, b_out.reshape(1, 2),
      prelu_a.reshape(1, 1).astype(jnp.float32))


_NPS = 320
_NPAD = 10240
_CAP = 1536
_ECH = 4000
_GB = 128
_NB = _CAP // _GB
_NCH = 32
_W = 64


def _sc_mesh():
    return plsc.VectorSubcoreMesh(core_axis_name="c", subcore_axis_name="s")


_SC_CP = pltpu.CompilerParams(use_tc_tiling_on_sc=False,
                              needs_layout_passes=False)


def _embed_gather(node_emb, gidx_pad):
    @functools.partial(
        pl.kernel, mesh=_sc_mesh(), compiler_params=_SC_CP,
        out_type=jax.ShapeDtypeStruct((_NPAD, _D), jnp.float32),
        scratch_types=[
            pltpu.VMEM((80,), jnp.int32),
            pltpu.VMEM((80, _D), jnp.float32),
            pltpu.SemaphoreType.DMA,
        ],
    )
    def k(tab_hbm, idx_hbm, out_hbm, idx_v, rows_v, sem):
        wid = lax.axis_index("s") * 2 + lax.axis_index("c")
        base = wid * _NPS

        @pl.loop(0, 4)
        def _(j):
            b = base + j * 80
            pltpu.sync_copy(idx_hbm.at[pl.ds(b, 80)], idx_v)
            pltpu.async_copy(tab_hbm.at[idx_v], rows_v, sem).wait()
            pltpu.sync_copy(rows_v, out_hbm.at[pl.ds(b, 80)])

    return k(node_emb, gidx_pad)


def _filter_edges(src, dst, sgn):
    f32, i32 = jnp.float32, jnp.int32
    out_type = [
        jax.ShapeDtypeStruct((32, _CAP), i32),
        jax.ShapeDtypeStruct((32, _CAP * 16), i32),
        jax.ShapeDtypeStruct((_NPAD,), f32),
    ]

    @functools.partial(
        pl.kernel, mesh=_sc_mesh(), out_type=out_type,
        compiler_params=_SC_CP,
        scratch_types=[
            pltpu.VMEM((_ECH,), i32),
            pltpu.VMEM((_ECH,), i32),
            pltpu.VMEM((_ECH,), i32),
            pltpu.VMEM((4096,), i32),
            pltpu.VMEM((4096,), i32),
            pltpu.VMEM((_CAP,), i32),
            pltpu.VMEM((_CAP,), i32),
            pltpu.VMEM((_CAP * 16,), i32),
            pltpu.VMEM((_NPS + 16,), f32),
            pltpu.VMEM((16,), i32),
            pltpu.SMEM((1,), i32),
            pltpu.SMEM((1,), i32),
        ],
    )
    def k(src_hbm, dst_hbm, sgn_hbm, lei_hbm, lsp_hbm, deg_hbm,
          esrc, edst, esgn, eflat, eldv, leidx, ldst, ldsplat, dacc, mbuf,
          cnt_s, off_s):
        wid = lax.axis_index("s") * 2 + lax.axis_index("c")
        base = wid * _NPS
        iota = lax.iota(i32, 16)
        zeros16 = jnp.zeros((16,), f32)
        ones16 = jnp.ones((16,), f32)
        zi16 = jnp.zeros((16,), i32)
        lane0 = iota == 0

        @pl.loop(0, _CAP, step=16)
        def _(i):
            leidx[pl.ds(i, 16)] = zi16
            ldst[pl.ds(i, 16)] = jnp.full((16,), _NPS, i32)

        @pl.loop(0, _CAP * 16, step=16)
        def _(i):
            ldsplat[pl.ds(i, 16)] = jnp.full((16,), _NPS, i32)

        @pl.loop(0, _ECH, step=16)
        def _(i):
            eflat[pl.ds(i, 16)] = zi16
            eldv[pl.ds(i, 16)] = zi16

        @pl.loop(0, _NPS + 16, step=16)
        def _(i):
            dacc[pl.ds(i, 16)] = zeros16

        cnt_s[0] = jnp.int32(0)
        mbuf[...] = jnp.full((16,), 1, i32)

        @pl.loop(0, _E // _ECH)
        def _(ck):
            off = ck * _ECH
            pltpu.sync_copy(src_hbm.at[pl.ds(off, _ECH)], esrc)
            pltpu.sync_copy(dst_hbm.at[pl.ds(off, _ECH)], edst)
            pltpu.sync_copy(sgn_hbm.at[pl.ds(off, _ECH)], esgn)

            @pl.loop(0, _ECH, step=16)
            def _(i):
                off_s[0] = i
                ii = off_s[0]
                tmsk = mbuf[...] >= 1
                plsc.store_scatter(
                    eflat, [ii + iota],
                    esrc[pl.ds(i, 16)] + esgn[pl.ds(i, 16)] * _N, mask=tmsk)
                plsc.store_scatter(
                    eldv, [ii + iota],
                    edst[pl.ds(i, 16)] - base, mask=tmsk)

            @pl.loop(0, _ECH)
            def _(i):
                iv = jnp.full((16,), i, i32)
                ev = plsc.load_gather(eflat, [iv])
                ldv = plsc.load_gather(eldv, [iv])
                cnt = cnt_s[0]
                keep = (ldv >= 0) & (ldv < _NPS) & (cnt < _CAP)
                m = lane0 & keep
                posv = jnp.full((16,), cnt, i32)
                plsc.store_scatter(leidx, [posv], ev, mask=m)
                plsc.store_scatter(ldst, [posv], ldv, mask=m)
                plsc.store_scatter(ldsplat, [posv * 16 + iota], ldv,
                                  mask=keep)
                cnt_s[0] = cnt + jnp.sum(m.astype(i32))

        @pl.loop(0, _CAP)
        def _(i):
            iv = jnp.full((16,), i, i32)
            lv = plsc.load_gather(ldst, [iv])
            plsc.addupdate_scatter(dacc, [lv], ones16, mask=lane0)

        pltpu.sync_copy(dacc.at[pl.ds(0, _NPS)], deg_hbm.at[pl.ds(base, _NPS)])
        pltpu.sync_copy(leidx, lei_hbm.at[wid])
        pltpu.sync_copy(ldsplat, lsp_hbm.at[wid])

    return k(src, dst, sgn)


def _seg_accum(tab_flat, lei, lsp):
    f32, i32 = jnp.float32, jnp.int32
    out_type = [jax.ShapeDtypeStruct((_NPAD, _T * _F), f32) for _ in range(4)]

    @functools.partial(
        pl.kernel, mesh=_sc_mesh(), out_type=out_type,
        compiler_params=_SC_CP,
        scratch_types=[
            pltpu.VMEM((_CAP,), i32),
            pltpu.VMEM((_CAP * 16,), i32),
            pltpu.VMEM((2 * _GB,), i32),
            pltpu.VMEM((_NPS + 1, _W), f32),
            pltpu.VMEM((_NPS + 1, _W), f32),
            pltpu.VMEM((_NPS + 1, _W), f32),
            pltpu.VMEM((_NPS + 1, _W), f32),
            pltpu.VMEM((2 * _GB, _W), f32),
            pltpu.SemaphoreType.DMA,
            pltpu.SemaphoreType.DMA,
        ],
    )
    def k(tab_hbm, lei_hbm, lsp_hbm,
          s1_hbm, s2_hbm, mn_hbm, mx_hbm,
          leidx, ldsplat, idxbuf, s1a, s2a, mna, mxa, gbuf, sem0, sem1):
        sems = (sem0, sem1)
        wid = lax.axis_index("s") * 2 + lax.axis_index("c")
        base = wid * _NPS
        iota = lax.iota(i32, 16)
        zeros16 = jnp.zeros((16,), f32)

        pltpu.sync_copy(lei_hbm.at[wid], leidx)
        pltpu.sync_copy(lsp_hbm.at[wid], ldsplat)

        @pl.loop(0, _NCH)
        def _(c):
            @pl.loop(0, _NPS + 1)
            def _(r):
                for kk in range(_W // 16):
                    sl = pl.ds(kk * 16, 16)
                    s1a[r, sl] = zeros16
                    s2a[r, sl] = zeros16
                    mna[r, sl] = jnp.full((16,), 3.0e38, f32)
                    mxa[r, sl] = jnp.full((16,), -3.0e38, f32)

            @pl.loop(0, _NB, step=2)
            def _(b):
                for p in range(2):
                    @pl.loop(0, _GB, step=16)
                    def _(j):
                        ev = leidx[pl.ds((b + p) * _GB + j, 16)]
                        idxbuf[pl.ds(p * _GB + j, 16)] = ev * _NCH + c

                for p in range(2):
                    pltpu.make_async_copy(
                        tab_hbm.at[idxbuf.at[pl.ds(p * _GB, _GB)]],
                        gbuf.at[pl.ds(p * _GB, _GB)], sems[p]).start()

                for p in range(2):
                    pltpu.make_async_copy(
                        tab_hbm.at[idxbuf.at[pl.ds(p * _GB, _GB)]],
                        gbuf.at[pl.ds(p * _GB, _GB)], sems[p]).wait()

                    @pl.loop(0, _GB)
                    def _(i):
                        ldv = ldsplat[pl.ds(((b + p) * _GB + i) * 16, 16)]
                        for kk in range(_W // 16):
                            iok = iota + kk * 16
                            row = gbuf[p * _GB + i, pl.ds(kk * 16, 16)]
                            plsc.addupdate_scatter(s1a, [ldv, iok], row)
                            plsc.addupdate_scatter(s2a, [ldv, iok], row * row)
                            cmn = plsc.load_gather(mna, [ldv, iok])
                            plsc.store_scatter(mna, [ldv, iok],
                                               jnp.minimum(cmn, row))
                            cmx = plsc.load_gather(mxa, [ldv, iok])
                            plsc.store_scatter(mxa, [ldv, iok],
                                               jnp.maximum(cmx, row))

            cs = pl.ds(c * _W, _W)
            rs = pl.ds(0, _NPS)
            pltpu.sync_copy(s1a.at[rs], s1_hbm.at[pl.ds(base, _NPS), cs])
            pltpu.sync_copy(s2a.at[rs], s2_hbm.at[pl.ds(base, _NPS), cs])
            pltpu.sync_copy(mna.at[rs], mn_hbm.at[pl.ds(base, _NPS), cs])
            pltpu.sync_copy(mxa.at[rs], mx_hbm.at[pl.ds(base, _NPS), cs])

    return k(tab_flat, lei, lsp)


def kernel(global_idx, acts, sign, edge_index, batch, node_emb, edge_emb,
           W_act, b_act, We, be, Wpre, bpre, Wpost, bpost, Wlin, blin,
           gamma, beta, W_fc1, b_fc1, W_out, b_out, prelu_a):
    src = edge_index[0].astype(jnp.int32)
    dst = edge_index[1].astype(jnp.int32)
    sgn = sign.astype(jnp.int32)

    gidx_pad = jnp.concatenate([global_idx.astype(jnp.int32),
                                jnp.zeros((_NPAD - _N,), jnp.int32)])
    xe = _embed_gather(node_emb, gidx_pad)
    x = _k_x(xe, acts, W_act, b_act.reshape(1, _D))

    lei, lds, deg = _filter_edges(src, dst, sgn)
    deg2d = deg.reshape(-1, 1)

    for l in range(2):
        Wd = Wpre[l][:, :_F, :].transpose(1, 0, 2).reshape(_F, _T * _F)
        Ws = Wpre[l][:, _F:2 * _F, :].transpose(1, 0, 2).reshape(_F, _T * _F)
        Wpre_e = Wpre[l][:, 2 * _F:, :]
        bpre_f = bpre[l].reshape(1, _T * _F)
        C2 = _k_c2(edge_emb, We[l], be[l], Wpre_e)
        P, Tbl = _k_pre(x, Wd, Ws, bpre_f, C2)
        S1, S2, MN, MX = _seg_accum(
            Tbl.reshape(2 * _N * _NCH, _W), lei, lds)
        y, bnsum, bnssq = _k_post(x, P, S1, S2, MN, MX, deg2d,
                                  Wpost[l], bpost[l].reshape(1, _T * _FO),
                                  Wlin[l], blin[l].reshape(1, _D))
        x = _k_bn(y, bnsum, bnssq, gamma[l].reshape(1, _D),
                  beta[l].reshape(1, _D))

    pooled = _k_pool(x, batch.astype(jnp.int32).reshape(-1, 1))
    return _k_head(pooled, W_fc1, b_fc1, W_out, b_out, prelu_a)


# speedup vs baseline: 1.8570x; 1.0412x over previous
import functools

import jax
import jax.numpy as jnp
import numpy as np
from jax import lax
from jax.experimental import pallas as pl
from jax.experimental.pallas import tpu as pltpu
from jax.experimental.pallas import tpu_sc as plsc

_N = 10000
_E = 40000
_P919 = 919
_D = 512
_T = 4
_F = 512
_FO = 128
_EDIM = 50
_NG = 64
_DEG_HIST = np.array([0.0, 500.0, 1000.0, 1500.0, 2000.0, 1800.0, 1200.0, 800.0,
                      500.0, 300.0, 200.0, 100.0, 60.0, 30.0, 10.0])
_AVG_DEG_LOG = float((np.log(np.arange(_DEG_HIST.shape[0]) + 1.0) * _DEG_HIST).sum()
                     / _DEG_HIST.sum())

_PREC = jax.lax.Precision.DEFAULT


def _dot(a, b):
    return jax.lax.dot_general(a, b, (((1,), (0,)), ((), ())), precision=_PREC,
                               preferred_element_type=jnp.float32)


def _kx_body(xe_ref, acts_ref, wact_ref, bact_ref, o_ref):
    o_ref[...] = (xe_ref[...] + _dot(acts_ref[...], wact_ref[...])
                  + bact_ref[...])


def _k_x(xe, acts, W_act, b_act):
    nb = 1000
    return pl.pallas_call(
        _kx_body,
        grid=(_N // nb,),
        in_specs=[
            pl.BlockSpec((nb, _D), lambda i: (i, 0)),
            pl.BlockSpec((nb, 2), lambda i: (i, 0)),
            pl.BlockSpec((2, _D), lambda i: (0, 0)),
            pl.BlockSpec((1, _D), lambda i: (0, 0)),
        ],
        out_specs=pl.BlockSpec((nb, _D), lambda i: (i, 0)),
        out_shape=jax.ShapeDtypeStruct((_N, _D), jnp.float32),
    )(xe, acts, W_act, b_act)


def _kc2_body(ee_ref, we_ref, be_ref, wpe_ref, o_ref):
    e2 = _dot(ee_ref[...], we_ref[...]) + be_ref[...]
    for t in range(_T):
        o_ref[:, t * _F:(t + 1) * _F] = _dot(e2, wpe_ref[t])


def _k_c2(edge_emb, We_l, be_l, Wpre_e_l):
    return pl.pallas_call(
        _kc2_body,
        out_shape=jax.ShapeDtypeStruct((2, _T * _F), jnp.float32),
    )(edge_emb, We_l, be_l.reshape(1, _F), Wpre_e_l)


def _kpre_body(x_ref, wd_ref, ws_ref, bpre_ref, c2_ref, p_ref, t_ref):
    x = x_ref[...]
    p_ref[...] = _dot(x, wd_ref[...]) + bpre_ref[...]
    q = _dot(x, ws_ref[...])
    t_ref[0] = q + c2_ref[0:1, :]
    t_ref[1] = q + c2_ref[1:2, :]


def _k_pre(x, Wd, Ws, bpre_f, C2):
    nb = 400
    return pl.pallas_call(
        _kpre_body,
        grid=(_N // nb,),
        in_specs=[
            pl.BlockSpec((nb, _D), lambda i: (i, 0)),
            pl.BlockSpec((_D, _T * _F), lambda i: (0, 0)),
            pl.BlockSpec((_D, _T * _F), lambda i: (0, 0)),
            pl.BlockSpec((1, _T * _F), lambda i: (0, 0)),
            pl.BlockSpec((2, _T * _F), lambda i: (0, 0)),
        ],
        out_specs=[
            pl.BlockSpec((nb, _T * _F), lambda i: (i, 0)),
            pl.BlockSpec((2, nb, _T * _F), lambda i: (0, i, 0)),
        ],
        out_shape=[
            jax.ShapeDtypeStruct((_N, _T * _F), jnp.float32),
            jax.ShapeDtypeStruct((2, _N, _T * _F), jnp.float32),
        ],
    )(x, Wd, Ws, bpre_f, C2)


def _kpost_body(x_ref, p_ref, s1_ref, s2_ref, mn_ref, mx_ref, deg_ref,
                wpost_ref, bpost_ref, wlin_ref, blin_ref,
                y_ref, bnsum_ref, bnssq_ref):
    i = pl.program_id(0)
    deg = deg_ref[...]
    degc = jnp.maximum(deg, 1.0)
    has = deg > 0.0
    K = p_ref[...]
    m1 = s1_ref[...] / degc
    mean = jnp.where(has, K + m1, 0.0)
    var = jnp.maximum(s2_ref[...] / degc - m1 * m1, 0.0)
    std = jnp.sqrt(var + 1e-5)
    mn = jnp.where(has, K + mn_ref[...], 0.0)
    mx = jnp.where(has, K + mx_ref[...], 0.0)
    ld = jnp.log(degc + 1.0)
    s2c = ld * (1.0 / _AVG_DEG_LOG)
    s3c = _AVG_DEG_LOG / ld
    x = x_ref[...]
    o_parts = []
    for t in range(_T):
        sl = slice(t * _F, (t + 1) * _F)
        agg = jnp.concatenate([mean[:, sl], mn[:, sl], mx[:, sl], std[:, sl]],
                              axis=1)
        post_h = jnp.concatenate([x, agg, agg * s2c, agg * s3c], axis=1)
        o_parts.append(_dot(post_h, wpost_ref[t])
                       + bpost_ref[:, t * _FO:(t + 1) * _FO])
    y = _dot(jnp.concatenate(o_parts, axis=1), wlin_ref[...]) + blin_ref[...]
    y_ref[...] = y

    @pl.when(i == 0)
    def _():
        bnsum_ref[...] = jnp.zeros_like(bnsum_ref)
        bnssq_ref[...] = jnp.zeros_like(bnssq_ref)

    bnsum_ref[...] += jnp.sum(y, axis=0, keepdims=True)
    bnssq_ref[...] += jnp.sum(y * y, axis=0, keepdims=True)


def _k_post(x, P, S1, S2, MN, MX, deg2d, Wpost_l, bpost_f, Wlin_l, blin_f):
    nb = 200
    big = lambda: pl.BlockSpec((nb, _T * _F), lambda i: (i, 0))
    return pl.pallas_call(
        _kpost_body,
        grid=(_N // nb,),
        in_specs=[
            pl.BlockSpec((nb, _D), lambda i: (i, 0)),
            big(), big(), big(), big(), big(),
            pl.BlockSpec((nb, 1), lambda i: (i, 0)),
            pl.BlockSpec((_T, 13 * _F, _FO), lambda i: (0, 0, 0)),
            pl.BlockSpec((1, _T * _FO), lambda i: (0, 0)),
            pl.BlockSpec((_D, _D), lambda i: (0, 0)),
            pl.BlockSpec((1, _D), lambda i: (0, 0)),
        ],
        out_specs=[
            pl.BlockSpec((nb, _D), lambda i: (i, 0)),
            pl.BlockSpec((1, _D), lambda i: (0, 0)),
            pl.BlockSpec((1, _D), lambda i: (0, 0)),
        ],
        out_shape=[
            jax.ShapeDtypeStruct((_N, _D), jnp.float32),
            jax.ShapeDtypeStruct((1, _D), jnp.float32),
            jax.ShapeDtypeStruct((1, _D), jnp.float32),
        ],
    )(x, P, S1, S2, MN, MX, deg2d, Wpost_l, bpost_f, Wlin_l, blin_f)


def _kbn_body(y_ref, sum_ref, ssq_ref, gamma_ref, beta_ref, o_ref):
    m = sum_ref[...] * (1.0 / _N)
    var = ssq_ref[...] * (1.0 / _N) - m * m
    inv = jax.lax.rsqrt(var + 1e-5)
    o_ref[...] = jnp.maximum((y_ref[...] - m) * inv * gamma_ref[...]
                             + beta_ref[...], 0.0)


def _k_bn(y, bnsum, bnssq, gamma_f, beta_f):
    nb = 1000
    return pl.pallas_call(
        _kbn_body,
        grid=(_N // nb,),
        in_specs=[
            pl.BlockSpec((nb, _D), lambda i: (i, 0)),
            pl.BlockSpec((1, _D), lambda i: (0, 0)),
            pl.BlockSpec((1, _D), lambda i: (0, 0)),
            pl.BlockSpec((1, _D), lambda i: (0, 0)),
            pl.BlockSpec((1, _D), lambda i: (0, 0)),
        ],
        out_specs=pl.BlockSpec((nb, _D), lambda i: (i, 0)),
        out_shape=jax.ShapeDtypeStruct((_N, _D), jnp.float32),
    )(y, bnsum, bnssq, gamma_f, beta_f)


def _kpool_body(x_ref, b_ref, o_ref):
    i = pl.program_id(0)

    @pl.when(i == 0)
    def _():
        o_ref[...] = jnp.zeros_like(o_ref)

    b = b_ref[...]
    gid = jax.lax.broadcasted_iota(jnp.int32, (b.shape[0], _NG), 1)
    oh = (b == gid).astype(jnp.float32)
    o_ref[...] += jax.lax.dot_general(oh, x_ref[...], (((0,), (0,)), ((), ())),
                                      precision=_PREC,
                                      preferred_element_type=jnp.float32)


def _k_pool(x, batch2d):
    nb = 1000
    return pl.pallas_call(
        _kpool_body,
        grid=(_N // nb,),
        in_specs=[
            pl.BlockSpec((nb, _D), lambda i: (i, 0)),
            pl.BlockSpec((nb, 1), lambda i: (i, 0)),
        ],
        out_specs=pl.BlockSpec((_NG, _D), lambda i: (0, 0)),
        out_shape=jax.ShapeDtypeStruct((_NG, _D), jnp.float32),
    )(x, batch2d)


def _khead_body(p_ref, wf_ref, bf_ref, wo_ref, bo_ref, a_ref, o_ref):
    h = _dot(p_ref[...], wf_ref[...]) + bf_ref[...]
    a = a_ref[0, 0]
    h = jnp.where(h >= 0.0, h, a * h)
    logits = _dot(h, wo_ref[...]) + bo_ref[...]
    mx = jnp.max(logits, axis=1, keepdims=True)
    lse = jnp.log(jnp.sum(jnp.exp(logits - mx), axis=1, keepdims=True)) + mx
    o_ref[...] = logits - lse


def _k_head(pooled, W_fc1, b_fc1, W_out, b_out, prelu_a):
    return pl.pallas_call(
        _khead_body,
        out_shape=jax.ShapeDtypeStruct((_NG, 2), jnp.float32),
    )(pooled, W_fc1, b_fc1.reshape(1, 2 * _D), W_out, b_out.reshape(1, 2),
      prelu_a.reshape(1, 1).astype(jnp.float32))


_NPS = 320
_NPAD = 10240
_CAP = 1536
_ECH = 4000
_GB = 128
_NB = _CAP // _GB
_NCH = 32
_W = 64


def _sc_mesh():
    return plsc.VectorSubcoreMesh(core_axis_name="c", subcore_axis_name="s")


_SC_CP = pltpu.CompilerParams(use_tc_tiling_on_sc=False,
                              needs_layout_passes=False)


def _embed_gather(node_emb, gidx_pad):
    @functools.partial(
        pl.kernel, mesh=_sc_mesh(), compiler_params=_SC_CP,
        out_type=jax.ShapeDtypeStruct((_NPAD, _D), jnp.float32),
        scratch_types=[
            pltpu.VMEM((80,), jnp.int32),
            pltpu.VMEM((80, _D), jnp.float32),
            pltpu.SemaphoreType.DMA,
        ],
    )
    def k(tab_hbm, idx_hbm, out_hbm, idx_v, rows_v, sem):
        wid = lax.axis_index("s") * 2 + lax.axis_index("c")
        base = wid * _NPS

        @pl.loop(0, 4)
        def _(j):
            b = base + j * 80
            pltpu.sync_copy(idx_hbm.at[pl.ds(b, 80)], idx_v)
            pltpu.async_copy(tab_hbm.at[idx_v], rows_v, sem).wait()
            pltpu.sync_copy(rows_v, out_hbm.at[pl.ds(b, 80)])

    return k(node_emb, gidx_pad)


def _filter_edges(src, dst, sgn):
    f32, i32 = jnp.float32, jnp.int32
    out_type = [
        jax.ShapeDtypeStruct((32, _CAP), i32),
        jax.ShapeDtypeStruct((32, _CAP), i32),
        jax.ShapeDtypeStruct((_NPAD,), f32),
    ]

    @functools.partial(
        pl.kernel, mesh=_sc_mesh(), out_type=out_type,
        compiler_params=_SC_CP,
        scratch_types=[
            pltpu.VMEM((_ECH,), i32),
            pltpu.VMEM((_ECH,), i32),
            pltpu.VMEM((_ECH,), i32),
            pltpu.VMEM((4096,), i32),
            pltpu.VMEM((4096,), i32),
            pltpu.VMEM((_CAP,), i32),
            pltpu.VMEM((_CAP,), i32),
            pltpu.VMEM((_NPS + 16,), f32),
            pltpu.VMEM((16,), i32),
            pltpu.SMEM((1,), i32),
            pltpu.SMEM((1,), i32),
        ],
    )
    def k(src_hbm, dst_hbm, sgn_hbm, lei_hbm, lds_hbm, deg_hbm,
          esrc, edst, esgn, eflat, eldv, leidx, ldst, dacc, mbuf,
          cnt_s, off_s):
        wid = lax.axis_index("s") * 2 + lax.axis_index("c")
        base = wid * _NPS
        iota = lax.iota(i32, 16)
        zeros16 = jnp.zeros((16,), f32)
        ones16 = jnp.ones((16,), f32)
        zi16 = jnp.zeros((16,), i32)
        lane0 = iota == 0

        @pl.loop(0, _CAP, step=16)
        def _(i):
            leidx[pl.ds(i, 16)] = zi16
            ldst[pl.ds(i, 16)] = jnp.full((16,), _NPS, i32)

        @pl.loop(0, _ECH, step=16)
        def _(i):
            eflat[pl.ds(i, 16)] = zi16
            eldv[pl.ds(i, 16)] = zi16

        @pl.loop(0, _NPS + 16, step=16)
        def _(i):
            dacc[pl.ds(i, 16)] = zeros16

        cnt_s[0] = jnp.int32(0)
        mbuf[...] = jnp.full((16,), 1, i32)

        @pl.loop(0, _E // _ECH)
        def _(ck):
            off = ck * _ECH
            pltpu.sync_copy(src_hbm.at[pl.ds(off, _ECH)], esrc)
            pltpu.sync_copy(dst_hbm.at[pl.ds(off, _ECH)], edst)
            pltpu.sync_copy(sgn_hbm.at[pl.ds(off, _ECH)], esgn)

            @pl.loop(0, _ECH, step=16)
            def _(i):
                off_s[0] = i
                ii = off_s[0]
                tmsk = mbuf[...] >= 1
                plsc.store_scatter(
                    eflat, [ii + iota],
                    esrc[pl.ds(i, 16)] + esgn[pl.ds(i, 16)] * _N, mask=tmsk)
                plsc.store_scatter(
                    eldv, [ii + iota],
                    edst[pl.ds(i, 16)] - base, mask=tmsk)

            @pl.loop(0, _ECH)
            def _(i):
                iv = jnp.full((16,), i, i32)
                ev = plsc.load_gather(eflat, [iv])
                ldv = plsc.load_gather(eldv, [iv])
                cnt = cnt_s[0]
                keep = (ldv >= 0) & (ldv < _NPS) & (cnt < _CAP)
                m = lane0 & keep
                posv = jnp.full((16,), cnt, i32)
                plsc.store_scatter(leidx, [posv], ev, mask=m)
                plsc.store_scatter(ldst, [posv], ldv, mask=m)
                cnt_s[0] = cnt + jnp.sum(m.astype(i32))

        @pl.loop(0, _CAP)
        def _(i):
            iv = jnp.full((16,), i, i32)
            lv = plsc.load_gather(ldst, [iv])
            plsc.addupdate_scatter(dacc, [lv], ones16, mask=lane0)

        pltpu.sync_copy(dacc.at[pl.ds(0, _NPS)], deg_hbm.at[pl.ds(base, _NPS)])
        pltpu.sync_copy(leidx, lei_hbm.at[wid])
        pltpu.sync_copy(ldst, lds_hbm.at[wid])

    return k(src, dst, sgn)


def _seg_accum(tab_flat, lei, lds):
    f32, i32 = jnp.float32, jnp.int32
    out_type = [jax.ShapeDtypeStruct((_NPAD, _T * _F), f32) for _ in range(4)]

    @functools.partial(
        pl.kernel, mesh=_sc_mesh(), out_type=out_type,
        compiler_params=_SC_CP,
        scratch_types=[
            pltpu.VMEM((_CAP,), i32),
            pltpu.VMEM((_CAP,), i32),
            pltpu.VMEM((4 * _GB,), i32),
            pltpu.VMEM((_NPS + 1, _W), f32),
            pltpu.VMEM((_NPS + 1, _W), f32),
            pltpu.VMEM((_NPS + 1, _W), f32),
            pltpu.VMEM((_NPS + 1, _W), f32),
            pltpu.VMEM((4 * _GB, _W), f32),
            pltpu.SemaphoreType.DMA,
            pltpu.SemaphoreType.DMA,
            pltpu.SemaphoreType.DMA,
            pltpu.SemaphoreType.DMA,
        ],
    )
    def k(tab_hbm, lei_hbm, lds_hbm,
          s1_hbm, s2_hbm, mn_hbm, mx_hbm,
          leidx, ldst, idxbuf, s1a, s2a, mna, mxa, gbuf,
          sem0, sem1, sem2, sem3):
        sems = (sem0, sem1, sem2, sem3)
        wid = lax.axis_index("s") * 2 + lax.axis_index("c")
        base = wid * _NPS
        iota = lax.iota(i32, 16)
        zeros16 = jnp.zeros((16,), f32)

        pltpu.sync_copy(lei_hbm.at[wid], leidx)
        pltpu.sync_copy(lds_hbm.at[wid], ldst)

        @pl.loop(0, _NCH)
        def _(c):
            @pl.loop(0, _NPS + 1)
            def _(r):
                for kk in range(_W // 16):
                    sl = pl.ds(kk * 16, 16)
                    s1a[r, sl] = zeros16
                    s2a[r, sl] = zeros16
                    mna[r, sl] = jnp.full((16,), 3.0e38, f32)
                    mxa[r, sl] = jnp.full((16,), -3.0e38, f32)

            @pl.loop(0, _NB, step=4)
            def _(b):
                for p in range(4):
                    @pl.loop(0, _GB, step=16)
                    def _(j):
                        ev = leidx[pl.ds((b + p) * _GB + j, 16)]
                        idxbuf[pl.ds(p * _GB + j, 16)] = ev * _NCH + c

                for p in range(4):
                    pltpu.make_async_copy(
                        tab_hbm.at[idxbuf.at[pl.ds(p * _GB, _GB)]],
                        gbuf.at[pl.ds(p * _GB, _GB)], sems[p]).start()

                for p in range(4):
                    pltpu.make_async_copy(
                        tab_hbm.at[idxbuf.at[pl.ds(p * _GB, _GB)]],
                        gbuf.at[pl.ds(p * _GB, _GB)], sems[p]).wait()

                    @pl.loop(0, _GB)
                    def _(i):
                        bidx = jnp.full((16,), (b + p) * _GB + i, i32)
                        ldv = plsc.load_gather(ldst, [bidx])
                        for kk in range(_W // 16):
                            iok = iota + kk * 16
                            row = gbuf[p * _GB + i, pl.ds(kk * 16, 16)]
                            plsc.addupdate_scatter(s1a, [ldv, iok], row)
                            plsc.addupdate_scatter(s2a, [ldv, iok], row * row)
                            cmn = plsc.load_gather(mna, [ldv, iok])
                            plsc.store_scatter(mna, [ldv, iok],
                                               jnp.minimum(cmn, row))
                            cmx = plsc.load_gather(mxa, [ldv, iok])
                            plsc.store_scatter(mxa, [ldv, iok],
                                               jnp.maximum(cmx, row))

            cs = pl.ds(c * _W, _W)
            rs = pl.ds(0, _NPS)
            pltpu.sync_copy(s1a.at[rs], s1_hbm.at[pl.ds(base, _NPS), cs])
            pltpu.sync_copy(s2a.at[rs], s2_hbm.at[pl.ds(base, _NPS), cs])
            pltpu.sync_copy(mna.at[rs], mn_hbm.at[pl.ds(base, _NPS), cs])
            pltpu.sync_copy(mxa.at[rs], mx_hbm.at[pl.ds(base, _NPS), cs])

    return k(tab_flat, lei, lds)


def kernel(global_idx, acts, sign, edge_index, batch, node_emb, edge_emb,
           W_act, b_act, We, be, Wpre, bpre, Wpost, bpost, Wlin, blin,
           gamma, beta, W_fc1, b_fc1, W_out, b_out, prelu_a):
    src = edge_index[0].astype(jnp.int32)
    dst = edge_index[1].astype(jnp.int32)
    sgn = sign.astype(jnp.int32)

    gidx_pad = jnp.concatenate([global_idx.astype(jnp.int32),
                                jnp.zeros((_NPAD - _N,), jnp.int32)])
    xe = _embed_gather(node_emb, gidx_pad)
    x = _k_x(xe, acts, W_act, b_act.reshape(1, _D))

    lei, lds, deg = _filter_edges(src, dst, sgn)
    deg2d = deg.reshape(-1, 1)

    for l in range(2):
        Wd = Wpre[l][:, :_F, :].transpose(1, 0, 2).reshape(_F, _T * _F)
        Ws = Wpre[l][:, _F:2 * _F, :].transpose(1, 0, 2).reshape(_F, _T * _F)
        Wpre_e = Wpre[l][:, 2 * _F:, :]
        bpre_f = bpre[l].reshape(1, _T * _F)
        C2 = _k_c2(edge_emb, We[l], be[l], Wpre_e)
        P, Tbl = _k_pre(x, Wd, Ws, bpre_f, C2)
        S1, S2, MN, MX = _seg_accum(
            Tbl.reshape(2 * _N * _NCH, _W), lei, lds)
        y, bnsum, bnssq = _k_post(x, P, S1, S2, MN, MX, deg2d,
                                  Wpost[l], bpost[l].reshape(1, _T * _FO),
                                  Wlin[l], blin[l].reshape(1, _D))
        x = _k_bn(y, bnsum, bnssq, gamma[l].reshape(1, _D),
                  beta[l].reshape(1, _D))

    pooled = _k_pool(x, batch.astype(jnp.int32).reshape(-1, 1))
    return _k_head(pooled, W_fc1, b_fc1, W_out, b_out, prelu_a)


# speedup vs baseline: 2.1461x; 1.1557x over previous
import functools

import jax
import jax.numpy as jnp
import numpy as np
from jax import lax
from jax.experimental import pallas as pl
from jax.experimental.pallas import tpu as pltpu
from jax.experimental.pallas import tpu_sc as plsc

_N = 10000
_E = 40000
_P919 = 919
_D = 512
_T = 4
_F = 512
_FO = 128
_EDIM = 50
_NG = 64
_DEG_HIST = np.array([0.0, 500.0, 1000.0, 1500.0, 2000.0, 1800.0, 1200.0, 800.0,
                      500.0, 300.0, 200.0, 100.0, 60.0, 30.0, 10.0])
_AVG_DEG_LOG = float((np.log(np.arange(_DEG_HIST.shape[0]) + 1.0) * _DEG_HIST).sum()
                     / _DEG_HIST.sum())

_PREC = jax.lax.Precision.DEFAULT


def _dot(a, b):
    return jax.lax.dot_general(a, b, (((1,), (0,)), ((), ())), precision=_PREC,
                               preferred_element_type=jnp.float32)


def _kx_body(xe_ref, acts_ref, wact_ref, bact_ref, o_ref):
    o_ref[...] = (xe_ref[...] + _dot(acts_ref[...], wact_ref[...])
                  + bact_ref[...])


def _k_x(xe, acts, W_act, b_act):
    nb = 1000
    return pl.pallas_call(
        _kx_body,
        grid=(_N // nb,),
        in_specs=[
            pl.BlockSpec((nb, _D), lambda i: (i, 0)),
            pl.BlockSpec((nb, 2), lambda i: (i, 0)),
            pl.BlockSpec((2, _D), lambda i: (0, 0)),
            pl.BlockSpec((1, _D), lambda i: (0, 0)),
        ],
        out_specs=pl.BlockSpec((nb, _D), lambda i: (i, 0)),
        out_shape=jax.ShapeDtypeStruct((_N, _D), jnp.float32),
    )(xe, acts, W_act, b_act)


def _kc2_body(ee_ref, we_ref, be_ref, wpe_ref, o_ref):
    e2 = _dot(ee_ref[...], we_ref[...]) + be_ref[...]
    for t in range(_T):
        o_ref[:, t * _F:(t + 1) * _F] = _dot(e2, wpe_ref[t])


def _k_c2(edge_emb, We_l, be_l, Wpre_e_l):
    return pl.pallas_call(
        _kc2_body,
        out_shape=jax.ShapeDtypeStruct((2, _T * _F), jnp.float32),
    )(edge_emb, We_l, be_l.reshape(1, _F), Wpre_e_l)


def _kpre_body(x_ref, wd_ref, ws_ref, bpre_ref, c2_ref, p_ref, t_ref):
    x = x_ref[...]
    p_ref[...] = _dot(x, wd_ref[...]) + bpre_ref[...]
    q = _dot(x, ws_ref[...])
    t_ref[0] = q + c2_ref[0:1, :]
    t_ref[1] = q + c2_ref[1:2, :]


def _k_pre(x, Wd, Ws, bpre_f, C2):
    nb = 400
    return pl.pallas_call(
        _kpre_body,
        grid=(_N // nb,),
        in_specs=[
            pl.BlockSpec((nb, _D), lambda i: (i, 0)),
            pl.BlockSpec((_D, _T * _F), lambda i: (0, 0)),
            pl.BlockSpec((_D, _T * _F), lambda i: (0, 0)),
            pl.BlockSpec((1, _T * _F), lambda i: (0, 0)),
            pl.BlockSpec((2, _T * _F), lambda i: (0, 0)),
        ],
        out_specs=[
            pl.BlockSpec((nb, _T * _F), lambda i: (i, 0)),
            pl.BlockSpec((2, nb, _T * _F), lambda i: (0, i, 0)),
        ],
        out_shape=[
            jax.ShapeDtypeStruct((_N, _T * _F), jnp.float32),
            jax.ShapeDtypeStruct((2, _N, _T * _F), jnp.float32),
        ],
    )(x, Wd, Ws, bpre_f, C2)


def _kpost_body(x_ref, p_ref, s1_ref, s2_ref, mn_ref, mx_ref, deg_ref,
                wpost_ref, bpost_ref, wlin_ref, blin_ref,
                y_ref, bnsum_ref, bnssq_ref):
    i = pl.program_id(0)
    deg = deg_ref[...]
    degc = jnp.maximum(deg, 1.0)
    has = deg > 0.0
    K = p_ref[...]
    m1 = s1_ref[...] / degc
    mean = jnp.where(has, K + m1, 0.0)
    var = jnp.maximum(s2_ref[...] / degc - m1 * m1, 0.0)
    std = jnp.sqrt(var + 1e-5)
    mn = jnp.where(has, K + mn_ref[...], 0.0)
    mx = jnp.where(has, K + mx_ref[...], 0.0)
    ld = jnp.log(degc + 1.0)
    s2c = ld * (1.0 / _AVG_DEG_LOG)
    s3c = _AVG_DEG_LOG / ld
    x = x_ref[...]
    o_parts = []
    for t in range(_T):
        sl = slice(t * _F, (t + 1) * _F)
        agg = jnp.concatenate([mean[:, sl], mn[:, sl], mx[:, sl], std[:, sl]],
                              axis=1)
        post_h = jnp.concatenate([x, agg, agg * s2c, agg * s3c], axis=1)
        o_parts.append(_dot(post_h, wpost_ref[t])
                       + bpost_ref[:, t * _FO:(t + 1) * _FO])
    y = _dot(jnp.concatenate(o_parts, axis=1), wlin_ref[...]) + blin_ref[...]
    y_ref[...] = y

    @pl.when(i == 0)
    def _():
        bnsum_ref[...] = jnp.zeros_like(bnsum_ref)
        bnssq_ref[...] = jnp.zeros_like(bnssq_ref)

    bnsum_ref[...] += jnp.sum(y, axis=0, keepdims=True)
    bnssq_ref[...] += jnp.sum(y * y, axis=0, keepdims=True)


def _k_post(x, P, S1, S2, MN, MX, deg2d, Wpost_l, bpost_f, Wlin_l, blin_f):
    nb = 200
    big = lambda: pl.BlockSpec((nb, _T * _F), lambda i: (i, 0))
    return pl.pallas_call(
        _kpost_body,
        grid=(_N // nb,),
        in_specs=[
            pl.BlockSpec((nb, _D), lambda i: (i, 0)),
            big(), big(), big(), big(), big(),
            pl.BlockSpec((nb, 1), lambda i: (i, 0)),
            pl.BlockSpec((_T, 13 * _F, _FO), lambda i: (0, 0, 0)),
            pl.BlockSpec((1, _T * _FO), lambda i: (0, 0)),
            pl.BlockSpec((_D, _D), lambda i: (0, 0)),
            pl.BlockSpec((1, _D), lambda i: (0, 0)),
        ],
        out_specs=[
            pl.BlockSpec((nb, _D), lambda i: (i, 0)),
            pl.BlockSpec((1, _D), lambda i: (0, 0)),
            pl.BlockSpec((1, _D), lambda i: (0, 0)),
        ],
        out_shape=[
            jax.ShapeDtypeStruct((_N, _D), jnp.float32),
            jax.ShapeDtypeStruct((1, _D), jnp.float32),
            jax.ShapeDtypeStruct((1, _D), jnp.float32),
        ],
    )(x, P, S1, S2, MN, MX, deg2d, Wpost_l, bpost_f, Wlin_l, blin_f)


def _kbn_body(y_ref, sum_ref, ssq_ref, gamma_ref, beta_ref, o_ref):
    m = sum_ref[...] * (1.0 / _N)
    var = ssq_ref[...] * (1.0 / _N) - m * m
    inv = jax.lax.rsqrt(var + 1e-5)
    o_ref[...] = jnp.maximum((y_ref[...] - m) * inv * gamma_ref[...]
                             + beta_ref[...], 0.0)


def _k_bn(y, bnsum, bnssq, gamma_f, beta_f):
    nb = 1000
    return pl.pallas_call(
        _kbn_body,
        grid=(_N // nb,),
        in_specs=[
            pl.BlockSpec((nb, _D), lambda i: (i, 0)),
            pl.BlockSpec((1, _D), lambda i: (0, 0)),
            pl.BlockSpec((1, _D), lambda i: (0, 0)),
            pl.BlockSpec((1, _D), lambda i: (0, 0)),
            pl.BlockSpec((1, _D), lambda i: (0, 0)),
        ],
        out_specs=pl.BlockSpec((nb, _D), lambda i: (i, 0)),
        out_shape=jax.ShapeDtypeStruct((_N, _D), jnp.float32),
    )(y, bnsum, bnssq, gamma_f, beta_f)


def _kpool_body(x_ref, b_ref, o_ref):
    i = pl.program_id(0)

    @pl.when(i == 0)
    def _():
        o_ref[...] = jnp.zeros_like(o_ref)

    b = b_ref[...]
    gid = jax.lax.broadcasted_iota(jnp.int32, (b.shape[0], _NG), 1)
    oh = (b == gid).astype(jnp.float32)
    o_ref[...] += jax.lax.dot_general(oh, x_ref[...], (((0,), (0,)), ((), ())),
                                      precision=_PREC,
                                      preferred_element_type=jnp.float32)


def _k_pool(x, batch2d):
    nb = 1000
    return pl.pallas_call(
        _kpool_body,
        grid=(_N // nb,),
        in_specs=[
            pl.BlockSpec((nb, _D), lambda i: (i, 0)),
            pl.BlockSpec((nb, 1), lambda i: (i, 0)),
        ],
        out_specs=pl.BlockSpec((_NG, _D), lambda i: (0, 0)),
        out_shape=jax.ShapeDtypeStruct((_NG, _D), jnp.float32),
    )(x, batch2d)


def _khead_body(p_ref, wf_ref, bf_ref, wo_ref, bo_ref, a_ref, o_ref):
    h = _dot(p_ref[...], wf_ref[...]) + bf_ref[...]
    a = a_ref[0, 0]
    h = jnp.where(h >= 0.0, h, a * h)
    logits = _dot(h, wo_ref[...]) + bo_ref[...]
    mx = jnp.max(logits, axis=1, keepdims=True)
    lse = jnp.log(jnp.sum(jnp.exp(logits - mx), axis=1, keepdims=True)) + mx
    o_ref[...] = logits - lse


def _k_head(pooled, W_fc1, b_fc1, W_out, b_out, prelu_a):
    return pl.pallas_call(
        _khead_body,
        out_shape=jax.ShapeDtypeStruct((_NG, 2), jnp.float32),
    )(pooled, W_fc1, b_fc1.reshape(1, 2 * _D), W_out, b_out.reshape(1, 2),
      prelu_a.reshape(1, 1).astype(jnp.float32))


_NPS = 320
_NPAD = 10240
_CAP = 1536
_ECH = 4000
_GB = 128
_NB = _CAP // _GB
_NCH = 32
_W = 64


def _sc_mesh():
    return plsc.VectorSubcoreMesh(core_axis_name="c", subcore_axis_name="s")


_SC_CP = pltpu.CompilerParams(use_tc_tiling_on_sc=False,
                              needs_layout_passes=False)


def _embed_gather(node_emb, gidx_pad):
    @functools.partial(
        pl.kernel, mesh=_sc_mesh(), compiler_params=_SC_CP,
        out_type=jax.ShapeDtypeStruct((_NPAD, _D), jnp.float32),
        scratch_types=[
            pltpu.VMEM((80,), jnp.int32),
            pltpu.VMEM((80, _D), jnp.float32),
            pltpu.SemaphoreType.DMA,
        ],
    )
    def k(tab_hbm, idx_hbm, out_hbm, idx_v, rows_v, sem):
        wid = lax.axis_index("s") * 2 + lax.axis_index("c")
        base = wid * _NPS

        @pl.loop(0, 4)
        def _(j):
            b = base + j * 80
            pltpu.sync_copy(idx_hbm.at[pl.ds(b, 80)], idx_v)
            pltpu.async_copy(tab_hbm.at[idx_v], rows_v, sem).wait()
            pltpu.sync_copy(rows_v, out_hbm.at[pl.ds(b, 80)])

    return k(node_emb, gidx_pad)


def _filter_edges(src, dst, sgn):
    f32, i32 = jnp.float32, jnp.int32
    out_type = [
        jax.ShapeDtypeStruct((32, _CAP), i32),
        jax.ShapeDtypeStruct((32, _CAP), i32),
        jax.ShapeDtypeStruct((_NPAD,), f32),
    ]

    @functools.partial(
        pl.kernel, mesh=_sc_mesh(), out_type=out_type,
        compiler_params=_SC_CP,
        scratch_types=[
            pltpu.VMEM((_ECH,), i32),
            pltpu.VMEM((_ECH,), i32),
            pltpu.VMEM((_ECH,), i32),
            pltpu.VMEM((4096,), i32),
            pltpu.VMEM((4096,), i32),
            pltpu.VMEM((_CAP,), i32),
            pltpu.VMEM((_CAP,), i32),
            pltpu.VMEM((_NPS + 16,), f32),
            pltpu.VMEM((16,), i32),
            pltpu.SMEM((1,), i32),
            pltpu.SMEM((1,), i32),
        ],
    )
    def k(src_hbm, dst_hbm, sgn_hbm, lei_hbm, lds_hbm, deg_hbm,
          esrc, edst, esgn, eflat, eldv, leidx, ldst, dacc, mbuf,
          cnt_s, off_s):
        wid = lax.axis_index("s") * 2 + lax.axis_index("c")
        base = wid * _NPS
        iota = lax.iota(i32, 16)
        zeros16 = jnp.zeros((16,), f32)
        ones16 = jnp.ones((16,), f32)
        zi16 = jnp.zeros((16,), i32)
        lane0 = iota == 0

        @pl.loop(0, _CAP, step=16)
        def _(i):
            leidx[pl.ds(i, 16)] = zi16
            ldst[pl.ds(i, 16)] = jnp.full((16,), _NPS, i32)

        @pl.loop(0, _ECH, step=16)
        def _(i):
            eflat[pl.ds(i, 16)] = zi16
            eldv[pl.ds(i, 16)] = zi16

        @pl.loop(0, _NPS + 16, step=16)
        def _(i):
            dacc[pl.ds(i, 16)] = zeros16

        cnt_s[0] = jnp.int32(0)
        mbuf[...] = jnp.full((16,), 1, i32)

        @pl.loop(0, _E // _ECH)
        def _(ck):
            off = ck * _ECH
            pltpu.sync_copy(src_hbm.at[pl.ds(off, _ECH)], esrc)
            pltpu.sync_copy(dst_hbm.at[pl.ds(off, _ECH)], edst)
            pltpu.sync_copy(sgn_hbm.at[pl.ds(off, _ECH)], esgn)

            @pl.loop(0, _ECH, step=16)
            def _(i):
                off_s[0] = i
                ii = off_s[0]
                tmsk = mbuf[...] >= 1
                plsc.store_scatter(
                    eflat, [ii + iota],
                    esrc[pl.ds(i, 16)] + esgn[pl.ds(i, 16)] * _N, mask=tmsk)
                plsc.store_scatter(
                    eldv, [ii + iota],
                    edst[pl.ds(i, 16)] - base, mask=tmsk)

            @pl.loop(0, _ECH)
            def _(i):
                iv = jnp.full((16,), i, i32)
                ev = plsc.load_gather(eflat, [iv])
                ldv = plsc.load_gather(eldv, [iv])
                cnt = cnt_s[0]
                keep = (ldv >= 0) & (ldv < _NPS) & (cnt < _CAP)
                m = lane0 & keep
                posv = jnp.full((16,), cnt, i32)
                plsc.store_scatter(leidx, [posv], ev, mask=m)
                plsc.store_scatter(ldst, [posv], ldv, mask=m)
                cnt_s[0] = cnt + jnp.sum(m.astype(i32))

        @pl.loop(0, _CAP)
        def _(i):
            iv = jnp.full((16,), i, i32)
            lv = plsc.load_gather(ldst, [iv])
            plsc.addupdate_scatter(dacc, [lv], ones16, mask=lane0)

        pltpu.sync_copy(dacc.at[pl.ds(0, _NPS)], deg_hbm.at[pl.ds(base, _NPS)])
        pltpu.sync_copy(leidx, lei_hbm.at[wid])
        pltpu.sync_copy(ldst, lds_hbm.at[wid])

    return k(src, dst, sgn)


def _seg_accum(tab_flat, lei, lds):
    f32, i32 = jnp.float32, jnp.int32
    out_type = [jax.ShapeDtypeStruct((_NPAD, _T * _F), f32) for _ in range(4)]

    @functools.partial(
        pl.kernel, mesh=_sc_mesh(), out_type=out_type,
        compiler_params=_SC_CP,
        scratch_types=[
            pltpu.VMEM((_CAP,), i32),
            pltpu.VMEM((_CAP,), i32),
            pltpu.VMEM((4 * _GB,), i32),
            pltpu.VMEM((_NPS + 1, _W), f32),
            pltpu.VMEM((_NPS + 1, _W), f32),
            pltpu.VMEM((_NPS + 1, _W), f32),
            pltpu.VMEM((_NPS + 1, _W), f32),
            pltpu.VMEM((4 * _GB, _W), f32),
            pltpu.SemaphoreType.DMA,
            pltpu.SemaphoreType.DMA,
            pltpu.SemaphoreType.DMA,
            pltpu.SemaphoreType.DMA,
        ],
    )
    def k(tab_hbm, lei_hbm, lds_hbm,
          s1_hbm, s2_hbm, mn_hbm, mx_hbm,
          leidx, ldst, idxbuf, s1a, s2a, mna, mxa, gbuf,
          sem0, sem1, sem2, sem3):
        sems = (sem0, sem1, sem2, sem3)
        wid = lax.axis_index("s") * 2 + lax.axis_index("c")
        base = wid * _NPS
        iota = lax.iota(i32, 16)
        zeros16 = jnp.zeros((16,), f32)

        pltpu.sync_copy(lei_hbm.at[wid], leidx)
        pltpu.sync_copy(lds_hbm.at[wid], ldst)

        @pl.loop(0, _NCH)
        def _(c):
            @pl.loop(0, _NPS + 1)
            def _(r):
                for kk in range(_W // 16):
                    sl = pl.ds(kk * 16, 16)
                    s1a[r, sl] = zeros16
                    s2a[r, sl] = zeros16
                    mna[r, sl] = jnp.full((16,), 3.0e38, f32)
                    mxa[r, sl] = jnp.full((16,), -3.0e38, f32)

            @pl.loop(0, _NB, step=4)
            def _(b):
                for p in range(4):
                    @pl.loop(0, _GB, step=16)
                    def _(j):
                        ev = leidx[pl.ds((b + p) * _GB + j, 16)]
                        idxbuf[pl.ds(p * _GB + j, 16)] = ev * _NCH + c

                for p in range(4):
                    pltpu.make_async_copy(
                        tab_hbm.at[idxbuf.at[pl.ds(p * _GB, _GB)]],
                        gbuf.at[pl.ds(p * _GB, _GB)], sems[p]).start()

                for p in range(4):
                    pltpu.make_async_copy(
                        tab_hbm.at[idxbuf.at[pl.ds(p * _GB, _GB)]],
                        gbuf.at[pl.ds(p * _GB, _GB)], sems[p]).wait()

                    @pl.loop(0, _GB)
                    def _(i):
                        bidx = jnp.full((16,), (b + p) * _GB + i, i32)
                        ldv = plsc.load_gather(ldst, [bidx])
                        r = jnp.max(ldv)
                        for kk in range(_W // 16):
                            sl = pl.ds(kk * 16, 16)
                            row = gbuf[p * _GB + i, sl]
                            s1a[r, sl] += row
                            s2a[r, sl] += row * row
                            mna[r, sl] = jnp.minimum(mna[r, sl], row)
                            mxa[r, sl] = jnp.maximum(mxa[r, sl], row)

            cs = pl.ds(c * _W, _W)
            rs = pl.ds(0, _NPS)
            pltpu.sync_copy(s1a.at[rs], s1_hbm.at[pl.ds(base, _NPS), cs])
            pltpu.sync_copy(s2a.at[rs], s2_hbm.at[pl.ds(base, _NPS), cs])
            pltpu.sync_copy(mna.at[rs], mn_hbm.at[pl.ds(base, _NPS), cs])
            pltpu.sync_copy(mxa.at[rs], mx_hbm.at[pl.ds(base, _NPS), cs])

    return k(tab_flat, lei, lds)


def kernel(global_idx, acts, sign, edge_index, batch, node_emb, edge_emb,
           W_act, b_act, We, be, Wpre, bpre, Wpost, bpost, Wlin, blin,
           gamma, beta, W_fc1, b_fc1, W_out, b_out, prelu_a):
    src = edge_index[0].astype(jnp.int32)
    dst = edge_index[1].astype(jnp.int32)
    sgn = sign.astype(jnp.int32)

    gidx_pad = jnp.concatenate([global_idx.astype(jnp.int32),
                                jnp.zeros((_NPAD - _N,), jnp.int32)])
    xe = _embed_gather(node_emb, gidx_pad)
    x = _k_x(xe, acts, W_act, b_act.reshape(1, _D))

    lei, lds, deg = _filter_edges(src, dst, sgn)
    deg2d = deg.reshape(-1, 1)

    for l in range(2):
        Wd = Wpre[l][:, :_F, :].transpose(1, 0, 2).reshape(_F, _T * _F)
        Ws = Wpre[l][:, _F:2 * _F, :].transpose(1, 0, 2).reshape(_F, _T * _F)
        Wpre_e = Wpre[l][:, 2 * _F:, :]
        bpre_f = bpre[l].reshape(1, _T * _F)
        C2 = _k_c2(edge_emb, We[l], be[l], Wpre_e)
        P, Tbl = _k_pre(x, Wd, Ws, bpre_f, C2)
        S1, S2, MN, MX = _seg_accum(
            Tbl.reshape(2 * _N * _NCH, _W), lei, lds)
        y, bnsum, bnssq = _k_post(x, P, S1, S2, MN, MX, deg2d,
                                  Wpost[l], bpost[l].reshape(1, _T * _FO),
                                  Wlin[l], blin[l].reshape(1, _D))
        x = _k_bn(y, bnsum, bnssq, gamma[l].reshape(1, _D),
                  beta[l].reshape(1, _D))

    pooled = _k_pool(x, batch.astype(jnp.int32).reshape(-1, 1))
    return _k_head(pooled, W_fc1, b_fc1, W_out, b_out, prelu_a)
